# baseline (speedup 1.0000x reference)
import functools

import jax
import jax.numpy as jnp
from jax import lax
from jax.experimental import pallas as pl
from jax.experimental.pallas import tpu as pltpu
from jax.experimental.pallas import tpu_sc as plsc

_NC = 2
_NS = 16
_NW = _NC * _NS
_L = 16


def _lr_body(xc_hbm, xn_hbm, wb_hbm, tbl_hbm, out_hbm,
             idx_v, fidx_v, g_v, xn_v, wb_v, out_v, sem,
             *, bpw, num_fields, num_dim):
  wid = lax.axis_index("s") * _NC + lax.axis_index("c")
  base = wid * bpw

  pltpu.sync_copy(xc_hbm.at[pl.ds(base, bpw), :], idx_v)
  pltpu.sync_copy(xn_hbm.at[pl.ds(base, bpw), :], xn_v)
  pltpu.sync_copy(wb_hbm, wb_v)

  iota = lax.iota(jnp.int32, _L)
  nchunks = bpw // _L

  def fire_body(f, _):
    def j_body(j, _):
      rows = j * _L + iota
      fidx_v[f, pl.ds(j * _L, _L)] = plsc.load_gather(
          idx_v, [rows, jnp.full((_L,), f, jnp.int32)])
      return 0

    lax.fori_loop(0, nchunks, j_body, 0)
    pltpu.async_copy(
        tbl_hbm.at[f].at[fidx_v.at[f]], g_v.at[f], sem)
    return 0

  lax.fori_loop(0, num_fields, fire_body, 0)

  def drain_body(f, _):
    pltpu.make_async_copy(
        tbl_hbm.at[f].at[fidx_v.at[f]], g_v.at[f], sem).wait()
    return 0

  lax.fori_loop(0, num_fields, drain_body, 0)

  bias_vec = wb_v[num_dim]

  def chunk_body(j, _):
    rows = j * _L + iota

    def f_body(f, acc):
      return acc + g_v[f, pl.ds(j * _L, _L)]

    acc = lax.fori_loop(0, num_fields, f_body, bias_vec)

    def d_body(d, acc):
      return acc + plsc.load_gather(
          xn_v, [rows, jnp.full((_L,), d, jnp.int32)]) * wb_v[d]

    acc = lax.fori_loop(0, num_dim, d_body, acc)
    out_v[pl.ds(j * _L, _L)] = acc
    return 0

  lax.fori_loop(0, nchunks, chunk_body, 0)
  pltpu.sync_copy(out_v, out_hbm.at[pl.ds(base, bpw)])


@functools.partial(jax.jit, static_argnames=())
def kernel(x_cat, x_num, tables, W, bias):
  B, F = x_cat.shape
  _, D_NUM = x_num.shape
  bpw = B // _NW

  wb = jnp.concatenate(
      [jnp.broadcast_to(W.reshape(D_NUM, 1), (D_NUM, _L)),
       jnp.broadcast_to(bias.reshape(1, 1), (1, _L))], axis=0)

  mesh = plsc.VectorSubcoreMesh(core_axis_name="c", subcore_axis_name="s",
                                num_cores=_NC, num_subcores=_NS)
  tbl2 = tables.reshape(F, tables.shape[1])
  body = functools.partial(_lr_body, bpw=bpw, num_fields=F, num_dim=D_NUM)
  out = pl.kernel(
      body,
      out_type=jax.ShapeDtypeStruct((B,), jnp.float32),
      mesh=mesh,
      compiler_params=pltpu.CompilerParams(needs_layout_passes=False,
                                           use_tc_tiling_on_sc=False),
      scratch_types=[
          pltpu.VMEM((bpw, F), jnp.int32),
          pltpu.VMEM((F, bpw), jnp.int32),
          pltpu.VMEM((F, bpw), jnp.float32),
          pltpu.VMEM((bpw, D_NUM), jnp.float32),
          pltpu.VMEM((D_NUM + 1, _L), jnp.float32),
          pltpu.VMEM((bpw,), jnp.float32),
          pltpu.SemaphoreType.DMA,
      ],
  )(x_cat, x_num, wb, tbl2)
  return out.reshape(B, 1)


# speedup vs baseline: 1.1676x; 1.1676x over previous
import functools

import jax
import jax.numpy as jnp
from jax import lax
from jax.experimental import pallas as pl
from jax.experimental.pallas import tpu as pltpu
from jax.experimental.pallas import tpu_sc as plsc

_NC = 2
_NS = 16
_NW = _NC * _NS
_L = 16


def _lr_body(idx_hbm, xn_hbm, wb_hbm, tbl_hbm, out_hbm,
             idx_v, g_v, xn_v, wb_v, out_v, sem,
             *, bpw, num_fields, num_dim):
  wid = lax.axis_index("s") * _NC + lax.axis_index("c")

  pltpu.sync_copy(idx_hbm.at[wid], idx_v)
  gather = pltpu.async_copy(tbl_hbm.at[idx_v], g_v, sem)
  pltpu.sync_copy(xn_hbm.at[wid], xn_v)
  pltpu.sync_copy(wb_hbm, wb_v)
  gather.wait()

  iota = lax.iota(jnp.int32, _L)
  iota_f = iota * num_fields
  iota_d = iota * num_dim
  bias_vec = wb_v[num_dim]

  def chunk_body(j, _):
    def f_body(f, acc):
      return acc + plsc.load_gather(g_v, [iota_f + (j * _L * num_fields + f)])

    acc = lax.fori_loop(0, num_fields, f_body, bias_vec)

    def d_body(d, acc):
      return acc + plsc.load_gather(
          xn_v, [iota_d + (j * _L * num_dim + d)]) * wb_v[d]

    acc = lax.fori_loop(0, num_dim, d_body, acc)
    out_v[pl.ds(j * _L, _L)] = acc
    return 0

  lax.fori_loop(0, bpw // _L, chunk_body, 0)
  pltpu.sync_copy(out_v, out_hbm.at[pl.ds(wid * bpw, bpw)])


@functools.partial(jax.jit, static_argnames=())
def kernel(x_cat, x_num, tables, W, bias):
  B, F = x_cat.shape
  _, D_NUM = x_num.shape
  V = tables.shape[1]
  bpw = B // _NW

  vp = (V + 127) // 128 * 128
  idx = (x_cat + (jnp.arange(F, dtype=jnp.int32) * vp)[None, :]).reshape(
      _NW, bpw * F)
  xn = x_num.reshape(_NW, bpw * D_NUM)
  tbl = jnp.pad(tables, ((0, 0), (0, vp - V), (0, 0))).reshape(F * vp)
  wb = jnp.concatenate(
      [jnp.broadcast_to(W.reshape(D_NUM, 1), (D_NUM, _L)),
       jnp.broadcast_to(bias.reshape(1, 1), (1, _L))], axis=0)

  mesh = plsc.VectorSubcoreMesh(core_axis_name="c", subcore_axis_name="s",
                                num_cores=_NC, num_subcores=_NS)
  body = functools.partial(_lr_body, bpw=bpw, num_fields=F, num_dim=D_NUM)
  out = pl.kernel(
      body,
      out_type=jax.ShapeDtypeStruct((B,), jnp.float32),
      mesh=mesh,
      compiler_params=pltpu.CompilerParams(needs_layout_passes=False),
      scratch_types=[
          pltpu.VMEM((bpw * F,), jnp.int32),
          pltpu.VMEM((bpw * F,), jnp.float32),
          pltpu.VMEM((bpw * D_NUM,), jnp.float32),
          pltpu.VMEM((D_NUM + 1, _L), jnp.float32),
          pltpu.VMEM((bpw,), jnp.float32),
          pltpu.SemaphoreType.DMA,
      ],
  )(idx, xn, wb, tbl)
  return out.reshape(B, 1)


# speedup vs baseline: 1.8727x; 1.6039x over previous
import functools

import jax
import jax.numpy as jnp
from jax import lax
from jax.experimental import pallas as pl
from jax.experimental.pallas import tpu as pltpu
from jax.experimental.pallas import tpu_sc as plsc

_NC = 2
_NS = 16
_NW = _NC * _NS
_L = 16


def _lr_body(*refs, bpw, num_fields, num_dim):
  idx_hbm, xn_hbm, wb_hbm = refs[:3]
  tbl_refs = refs[3:3 + num_fields]
  out_hbm = refs[3 + num_fields]
  rest = refs[4 + num_fields:]
  idx_v, xn_v, wb_v, out_v = rest[:4]
  fidx_bufs = rest[4:4 + num_fields]
  g_bufs = rest[4 + num_fields:4 + 2 * num_fields]
  sem = rest[4 + 2 * num_fields]

  wid = lax.axis_index("s") * _NC + lax.axis_index("c")

  pltpu.sync_copy(idx_hbm.at[wid], idx_v)
  pltpu.sync_copy(xn_hbm.at[wid], xn_v)
  pltpu.sync_copy(wb_hbm, wb_v)

  iota = lax.iota(jnp.int32, _L)
  iota_f = iota * num_fields
  iota_d = iota * num_dim
  nchunks = bpw // _L

  for f in range(num_fields):
    def j_body(j, _, f=f):
      fidx_bufs[f][pl.ds(j * _L, _L)] = plsc.load_gather(
          idx_v, [iota_f + (j * _L * num_fields + f)])
      return 0

    lax.fori_loop(0, nchunks, j_body, 0)
    pltpu.async_copy(tbl_refs[f].at[fidx_bufs[f]], g_bufs[f], sem)

  for f in range(num_fields):
    pltpu.make_async_copy(
        tbl_refs[f].at[fidx_bufs[f]], g_bufs[f], sem).wait()

  bias_vec = wb_v[num_dim]

  def chunk_body(j, _):
    acc = bias_vec
    for f in range(num_fields):
      acc = acc + g_bufs[f][pl.ds(j * _L, _L)]

    def d_body(d, acc):
      return acc + plsc.load_gather(
          xn_v, [iota_d + (j * _L * num_dim + d)]) * wb_v[d]

    acc = lax.fori_loop(0, num_dim, d_body, acc)
    out_v[pl.ds(j * _L, _L)] = acc
    return 0

  lax.fori_loop(0, bpw // _L, chunk_body, 0)
  pltpu.sync_copy(out_v, out_hbm.at[pl.ds(wid * bpw, bpw)])


@functools.partial(jax.jit, static_argnames=())
def kernel(x_cat, x_num, tables, W, bias):
  B, F = x_cat.shape
  _, D_NUM = x_num.shape
  bpw = B // _NW

  idx = x_cat.reshape(_NW, bpw * F)
  xn = x_num.reshape(_NW, bpw * D_NUM)
  tbl_slices = [tables[f, :, 0] for f in range(F)]
  wb = jnp.concatenate(
      [jnp.broadcast_to(W.reshape(D_NUM, 1), (D_NUM, _L)),
       jnp.broadcast_to(bias.reshape(1, 1), (1, _L))], axis=0)

  mesh = plsc.VectorSubcoreMesh(core_axis_name="c", subcore_axis_name="s",
                                num_cores=_NC, num_subcores=_NS)
  body = functools.partial(_lr_body, bpw=bpw, num_fields=F, num_dim=D_NUM)
  out = pl.kernel(
      body,
      out_type=jax.ShapeDtypeStruct((B,), jnp.float32),
      mesh=mesh,
      compiler_params=pltpu.CompilerParams(needs_layout_passes=False),
      scratch_types=(
          [pltpu.VMEM((bpw * F,), jnp.int32),
           pltpu.VMEM((bpw * D_NUM,), jnp.float32),
           pltpu.VMEM((D_NUM + 1, _L), jnp.float32),
           pltpu.VMEM((bpw,), jnp.float32)]
          + [pltpu.VMEM((bpw,), jnp.int32) for _ in range(F)]
          + [pltpu.VMEM((bpw,), jnp.float32) for _ in range(F)]
          + [pltpu.SemaphoreType.DMA]
      ),
  )(idx, xn, wb, *tbl_slices)
  return out.reshape(B, 1)
